# Initial kernel scaffold; baseline (speedup 1.0000x reference)
import jax
import jax.numpy as jnp
from jax import lax
from jax.experimental import pallas as pl
from jax.experimental.pallas import tpu as pltpu
from jax.experimental.pallas import tpu_sc as plsc

N_NODES = 10000
N_EDGES = 320000
D = 128
CW = 16

NC = 2
NS = 16
NW = NC * NS
EPW = N_EDGES // NW
K = 80
CHUNKS = EPW // K
RPT = N_NODES // NS

_mesh = plsc.VectorSubcoreMesh(core_axis_name="c", subcore_axis_name="s")


def _make_agg(with_counts: bool):
    out_type = [jax.ShapeDtypeStruct((NC, N_NODES, D), jnp.float32)]
    scratch = [
        pltpu.VMEM((K,), jnp.int32),
        pltpu.VMEM((K,), jnp.int32),
        pltpu.VMEM((K, D), jnp.float32),
        pltpu.VMEM_SHARED((N_NODES, D), jnp.float32),
    ]
    if with_counts:
        out_type.append(jax.ShapeDtypeStruct((NC, N_NODES, CW), jnp.float32))
        scratch += [
            pltpu.VMEM((K, CW), jnp.float32),
            pltpu.VMEM_SHARED((N_NODES, CW), jnp.float32),
        ]

    def body(h_hbm, src_hbm, dst_hbm, zd_hbm, zc_hbm, ones_hbm, *rest):
        if with_counts:
            (p_hbm, pc_hbm, idx_s, idx_d, rows, acc, ones_v, cacc) = rest
        else:
            (p_hbm, idx_s, idx_d, rows, acc) = rest
        c = lax.axis_index("c")
        s = lax.axis_index("s")
        wid = c * NS + s
        rbase = s * RPT

        pltpu.sync_copy(zd_hbm, acc.at[pl.ds(rbase, RPT)])
        if with_counts:
            pltpu.sync_copy(zc_hbm, cacc.at[pl.ds(rbase, RPT)])
            pltpu.sync_copy(ones_hbm, ones_v)
        plsc.subcore_barrier()

        ebase = wid * EPW

        @pl.loop(0, CHUNKS)
        def _(i):
            off = pl.multiple_of(ebase + i * K, 8)
            pltpu.sync_copy(src_hbm.at[pl.ds(off, K)], idx_s)
            pltpu.sync_copy(dst_hbm.at[pl.ds(off, K)], idx_d)
            pltpu.sync_copy(h_hbm.at[idx_s], rows)
            pltpu.sync_copy(rows, acc.at[idx_d], add=True)
            if with_counts:
                pltpu.sync_copy(ones_v, cacc.at[idx_d], add=True)

        plsc.subcore_barrier()

        pltpu.sync_copy(acc.at[pl.ds(rbase, RPT)],
                        p_hbm.at[c].at[pl.ds(rbase, RPT)])
        if with_counts:
            pltpu.sync_copy(cacc.at[pl.ds(rbase, RPT)],
                            pc_hbm.at[c].at[pl.ds(rbase, RPT)])

    return pl.kernel(body, out_type=tuple(out_type) if with_counts else out_type[0],
                     mesh=_mesh, scratch_types=scratch)


_agg_counts = _make_agg(True)
_agg = _make_agg(False)


def _tc_layer1(p_ref, pc_ref, h_ref, wl_ref, wr_ref, b_ref,
               out_ref, recip_ref):
    cnt = pc_ref[0, :, 0:1] + pc_ref[1, :, 0:1]
    recip = 1.0 / jnp.maximum(cnt, 1.0)
    recip_b = jnp.broadcast_to(recip, (N_NODES, D))
    recip_ref[...] = recip_b
    mean = (p_ref[0] + p_ref[1]) * recip_b
    acc = jnp.dot(mean, wl_ref[...], preferred_element_type=jnp.float32)
    acc += jnp.dot(h_ref[...], wr_ref[...], preferred_element_type=jnp.float32)
    out_ref[...] = jnp.maximum(acc + b_ref[...], 0.0)


def _tc_layer(p_ref, recip_ref, h_ref, wl_ref, wr_ref, b_ref, out_ref):
    mean = (p_ref[0] + p_ref[1]) * recip_ref[...]
    acc = jnp.dot(mean, wl_ref[...], preferred_element_type=jnp.float32)
    acc += jnp.dot(h_ref[...], wr_ref[...], preferred_element_type=jnp.float32)
    out_ref[...] = jnp.maximum(acc + b_ref[...], 0.0)


def _tc_layer3(p_ref, recip_ref, h_ref, wl_ref, wr_ref, b_ref,
               wg_ref, bg_ref, out_ref):
    mean = (p_ref[0] + p_ref[1]) * recip_ref[...]
    acc = jnp.dot(mean, wl_ref[...], preferred_element_type=jnp.float32)
    acc += jnp.dot(h_ref[...], wr_ref[...], preferred_element_type=jnp.float32)
    h3 = jnp.maximum(acc + b_ref[...], 0.0)
    g = jnp.mean(h3, axis=0, keepdims=True)
    z = jnp.dot(g, wg_ref[...], preferred_element_type=jnp.float32) + bg_ref[...]
    out_ref[...] = 1.0 / (1.0 + jnp.exp(-z))


def kernel(x, edge_index, W1_l, W1_r, b1, W2_l, W2_r, b2, W3_l, W3_r, b3,
           Wg, bg):
    src = edge_index[0].astype(jnp.int32)
    dst = edge_index[1].astype(jnp.int32)
    zd = jnp.zeros((RPT, D), jnp.float32)
    zc = jnp.zeros((RPT, CW), jnp.float32)
    ones = jnp.ones((K, CW), jnp.float32)
    b1r = b1.reshape(1, D)
    b2r = b2.reshape(1, D)
    b3r = b3.reshape(1, D)
    bgr = bg.reshape(1, -1)

    p1, pc = _agg_counts(x, src, dst, zd, zc, ones)
    h1, recip = pl.pallas_call(
        _tc_layer1,
        out_shape=(jax.ShapeDtypeStruct((N_NODES, D), jnp.float32),
                   jax.ShapeDtypeStruct((N_NODES, D), jnp.float32)),
    )(p1, pc, x, W1_l, W1_r, b1r)

    p2 = _agg(h1, src, dst, zd, zc, ones)
    h2 = pl.pallas_call(
        _tc_layer,
        out_shape=jax.ShapeDtypeStruct((N_NODES, D), jnp.float32),
    )(p2, recip, h1, W2_l, W2_r, b2r)

    p3 = _agg(h2, src, dst, zd, zc, ones)
    out = pl.pallas_call(
        _tc_layer3,
        out_shape=jax.ShapeDtypeStruct((1, Wg.shape[1]), jnp.float32),
    )(p3, recip, h2, W3_l, W3_r, b3r, Wg, bgr)
    return out


# speedup vs baseline: 4.2435x; 4.2435x over previous
import jax
import jax.numpy as jnp
from jax import lax
from jax.experimental import pallas as pl
from jax.experimental.pallas import tpu as pltpu
from jax.experimental.pallas import tpu_sc as plsc

N_NODES = 10000
N_EDGES = 320000
D = 128

NC = 2
NS = 16
NW = NC * NS
EPW = N_EDGES // NW
K = 80
CHUNKS = EPW // K
RB = 80
RCHUNKS = N_NODES // RB

_mesh = plsc.VectorSubcoreMesh(core_axis_name="c", subcore_axis_name="s")

_DEBUG_STAGE = 4


def _make_agg(with_gather: bool):
    scratch = [
        pltpu.VMEM((1, K), jnp.int32),
        pltpu.VMEM((K, D), jnp.float32),
        pltpu.VMEM((RB, D), jnp.float32),
        pltpu.VMEM_SHARED((N_NODES, D), jnp.float32),
    ]
    if with_gather:
        scratch.insert(0, pltpu.VMEM((1, K), jnp.int32))

    def body(h_hbm, src_hbm, dst_hbm, zd_hbm, *rest):
        if with_gather:
            (p_hbm, idx_s, idx_d, rows, zbuf, acc) = rest
        else:
            (p_hbm, idx_d, rows, zbuf, acc) = rest
        c = lax.axis_index("c")
        s = lax.axis_index("s")
        wid = c * NS + s

        pltpu.sync_copy(zd_hbm, zbuf)
        if not with_gather:
            pltpu.sync_copy(h_hbm.at[pl.ds(0, K)], rows)

        @pl.loop(s, RCHUNKS, step=NS)
        def _(j):
            rb = pl.multiple_of(j * RB, 8)
            pltpu.sync_copy(zbuf, acc.at[pl.ds(rb, RB)])

        plsc.subcore_barrier()

        cbase = wid * CHUNKS

        @pl.loop(0, CHUNKS)
        def _(i):
            if _DEBUG_STAGE >= 2:
                pltpu.sync_copy(dst_hbm.at[cbase + i], idx_d)
                if with_gather:
                    pltpu.sync_copy(src_hbm.at[cbase + i], idx_s)
                    pltpu.sync_copy(h_hbm.at[idx_s.at[0]], rows)
            if _DEBUG_STAGE >= 3:
                pltpu.sync_copy(rows, acc.at[idx_d.at[0]], add=True)

        plsc.subcore_barrier()

        obase = c * N_NODES

        @pl.loop(s, RCHUNKS, step=NS)
        def _(j):
            rb = pl.multiple_of(j * RB, 8)
            ob = pl.multiple_of(obase + j * RB, 8)
            pltpu.sync_copy(acc.at[pl.ds(rb, RB)], zbuf)
            pltpu.sync_copy(zbuf, p_hbm.at[pl.ds(ob, RB)])

    return pl.kernel(
        body,
        out_type=jax.ShapeDtypeStruct((NC * N_NODES, D), jnp.float32),
        mesh=_mesh, scratch_types=scratch)


_agg = _make_agg(True)
_counts = _make_agg(False)


def _tc_layer1(p_ref, pc_ref, h_ref, wl_ref, wr_ref, b_ref,
               out_ref, recip_ref):
    cnt = pc_ref[0, :, 0:1] + pc_ref[1, :, 0:1]
    recip = 1.0 / jnp.maximum(cnt, 1.0)
    recip_b = jnp.broadcast_to(recip, (N_NODES, D))
    recip_ref[...] = recip_b
    mean = (p_ref[0] + p_ref[1]) * recip_b
    acc = jnp.dot(mean, wl_ref[...], preferred_element_type=jnp.float32)
    acc += jnp.dot(h_ref[...], wr_ref[...], preferred_element_type=jnp.float32)
    out_ref[...] = jnp.maximum(acc + b_ref[...], 0.0)


def _tc_layer(p_ref, recip_ref, h_ref, wl_ref, wr_ref, b_ref, out_ref):
    mean = (p_ref[0] + p_ref[1]) * recip_ref[...]
    acc = jnp.dot(mean, wl_ref[...], preferred_element_type=jnp.float32)
    acc += jnp.dot(h_ref[...], wr_ref[...], preferred_element_type=jnp.float32)
    out_ref[...] = jnp.maximum(acc + b_ref[...], 0.0)


def _tc_layer3(p_ref, recip_ref, h_ref, wl_ref, wr_ref, b_ref,
               wg_ref, bg_ref, out_ref):
    mean = (p_ref[0] + p_ref[1]) * recip_ref[...]
    acc = jnp.dot(mean, wl_ref[...], preferred_element_type=jnp.float32)
    acc += jnp.dot(h_ref[...], wr_ref[...], preferred_element_type=jnp.float32)
    h3 = jnp.maximum(acc + b_ref[...], 0.0)
    g = jnp.mean(h3, axis=0, keepdims=True)
    z = jnp.dot(g, wg_ref[...], preferred_element_type=jnp.float32) + bg_ref[...]
    out_ref[...] = 1.0 / (1.0 + jnp.exp(-z))


def kernel(x, edge_index, W1_l, W1_r, b1, W2_l, W2_r, b2, W3_l, W3_r, b3,
           Wg, bg):
    src = edge_index[0].astype(jnp.int32).reshape(NW * CHUNKS, 1, K)
    dst = edge_index[1].astype(jnp.int32).reshape(NW * CHUNKS, 1, K)
    zd = jnp.zeros((RB, D), jnp.float32)
    ones = jnp.ones((K, D), jnp.float32)
    b1r = b1.reshape(1, D)
    b2r = b2.reshape(1, D)
    b3r = b3.reshape(1, D)
    bgr = bg.reshape(1, -1)

    pc = _counts(ones, src, dst, zd).reshape(NC, N_NODES, D)
    p1 = _agg(x, src, dst, zd).reshape(NC, N_NODES, D)
    h1, recip = pl.pallas_call(
        _tc_layer1,
        out_shape=(jax.ShapeDtypeStruct((N_NODES, D), jnp.float32),
                   jax.ShapeDtypeStruct((N_NODES, D), jnp.float32)),
    )(p1, pc, x, W1_l, W1_r, b1r)

    p2 = _agg(h1, src, dst, zd).reshape(NC, N_NODES, D)
    h2 = pl.pallas_call(
        _tc_layer,
        out_shape=jax.ShapeDtypeStruct((N_NODES, D), jnp.float32),
    )(p2, recip, h1, W2_l, W2_r, b2r)

    p3 = _agg(h2, src, dst, zd).reshape(NC, N_NODES, D)
    out = pl.pallas_call(
        _tc_layer3,
        out_shape=jax.ShapeDtypeStruct((1, Wg.shape[1]), jnp.float32),
    )(p3, recip, h2, W3_l, W3_r, b3r, Wg, bgr)
    return out


# speedup vs baseline: 7.1141x; 1.6765x over previous
import jax
import jax.numpy as jnp
from jax import lax
from jax.experimental import pallas as pl
from jax.experimental.pallas import tpu as pltpu
from jax.experimental.pallas import tpu_sc as plsc

N_NODES = 10000
N_EDGES = 320000
D = 128

NC = 2
NS = 16
NW = NC * NS
EPW = N_EDGES // NW
K = 100
CHUNKS = EPW // K
RB = 80
RCHUNKS = N_NODES // RB

_mesh = plsc.VectorSubcoreMesh(core_axis_name="c", subcore_axis_name="s")


def _make_agg(with_gather: bool):
    scratch = [
        pltpu.VMEM((2, K), jnp.int32),
        pltpu.VMEM((2, K), jnp.int32),
        pltpu.VMEM((K, D), jnp.float32),
        pltpu.VMEM((K, D), jnp.float32),
        pltpu.VMEM((RB, D), jnp.float32),
        pltpu.VMEM_SHARED((N_NODES, D), jnp.float32),
        pltpu.SemaphoreType.DMA,
        pltpu.SemaphoreType.DMA,
        pltpu.SemaphoreType.DMA,
        pltpu.SemaphoreType.DMA,
        pltpu.SemaphoreType.DMA,
        pltpu.SemaphoreType.DMA,
    ]

    def body(h_hbm, ei_hbm, zd_hbm, *rest):
        (p_hbm, idx0, idx1, rows0, rows1, zbuf, acc,
         sa0, sa1, sb0, sb1, sc0, sc1) = rest
        c = lax.axis_index("c")
        s = lax.axis_index("s")
        wid = c * NS + s

        pltpu.sync_copy(zd_hbm, zbuf)
        if not with_gather:
            pltpu.sync_copy(h_hbm.at[pl.ds(0, K)], rows0)
            pltpu.sync_copy(h_hbm.at[pl.ds(0, K)], rows1)

        @pl.loop(s, RCHUNKS, step=NS)
        def _(j):
            rb = pl.multiple_of(j * RB, 8)
            pltpu.sync_copy(zbuf, acc.at[pl.ds(rb, RB)])

        plsc.subcore_barrier()

        cbase = wid * CHUNKS

        @pl.loop(0, CHUNKS // 2)
        def _(g):
            j0 = cbase + 2 * g
            a0 = pltpu.async_copy(ei_hbm.at[j0], idx0, sa0)
            a1 = pltpu.async_copy(ei_hbm.at[j0 + 1], idx1, sa1)
            if with_gather:
                a0.wait()
                b0 = pltpu.async_copy(h_hbm.at[idx0.at[0]], rows0, sb0)
                a1.wait()
                b1 = pltpu.async_copy(h_hbm.at[idx1.at[0]], rows1, sb1)
                b0.wait()
                c0 = pltpu.async_copy(rows0, acc.at[idx0.at[1]], sc0,
                                      add=True)
                b1.wait()
                c1 = pltpu.async_copy(rows1, acc.at[idx1.at[1]], sc1,
                                      add=True)
            else:
                a0.wait()
                c0 = pltpu.async_copy(rows0, acc.at[idx0.at[1]], sc0,
                                      add=True)
                a1.wait()
                c1 = pltpu.async_copy(rows1, acc.at[idx1.at[1]], sc1,
                                      add=True)
            c0.wait()
            c1.wait()

        plsc.subcore_barrier()

        obase = c * N_NODES

        @pl.loop(s, RCHUNKS, step=NS)
        def _(j):
            rb = pl.multiple_of(j * RB, 8)
            ob = pl.multiple_of(obase + j * RB, 8)
            pltpu.sync_copy(acc.at[pl.ds(rb, RB)], zbuf)
            pltpu.sync_copy(zbuf, p_hbm.at[pl.ds(ob, RB)])

    return pl.kernel(
        body,
        out_type=jax.ShapeDtypeStruct((NC * N_NODES, D), jnp.float32),
        mesh=_mesh, scratch_types=scratch)


_agg = _make_agg(True)
_counts = _make_agg(False)


def _tc_layer1(p_ref, pc_ref, h_ref, wl_ref, wr_ref, b_ref,
               out_ref, recip_ref):
    cnt = pc_ref[0, :, 0:1] + pc_ref[1, :, 0:1]
    recip = 1.0 / jnp.maximum(cnt, 1.0)
    recip_b = jnp.broadcast_to(recip, (N_NODES, D))
    recip_ref[...] = recip_b
    mean = (p_ref[0] + p_ref[1]) * recip_b
    acc = jnp.dot(mean, wl_ref[...], preferred_element_type=jnp.float32)
    acc += jnp.dot(h_ref[...], wr_ref[...], preferred_element_type=jnp.float32)
    out_ref[...] = jnp.maximum(acc + b_ref[...], 0.0)


def _tc_layer(p_ref, recip_ref, h_ref, wl_ref, wr_ref, b_ref, out_ref):
    mean = (p_ref[0] + p_ref[1]) * recip_ref[...]
    acc = jnp.dot(mean, wl_ref[...], preferred_element_type=jnp.float32)
    acc += jnp.dot(h_ref[...], wr_ref[...], preferred_element_type=jnp.float32)
    out_ref[...] = jnp.maximum(acc + b_ref[...], 0.0)


def _tc_layer3(p_ref, recip_ref, h_ref, wl_ref, wr_ref, b_ref,
               wg_ref, bg_ref, out_ref):
    mean = (p_ref[0] + p_ref[1]) * recip_ref[...]
    acc = jnp.dot(mean, wl_ref[...], preferred_element_type=jnp.float32)
    acc += jnp.dot(h_ref[...], wr_ref[...], preferred_element_type=jnp.float32)
    h3 = jnp.maximum(acc + b_ref[...], 0.0)
    g = jnp.mean(h3, axis=0, keepdims=True)
    z = jnp.dot(g, wg_ref[...], preferred_element_type=jnp.float32) + bg_ref[...]
    out_ref[...] = 1.0 / (1.0 + jnp.exp(-z))


def kernel(x, edge_index, W1_l, W1_r, b1, W2_l, W2_r, b2, W3_l, W3_r, b3,
           Wg, bg):
    ei = jnp.stack([edge_index[0].astype(jnp.int32).reshape(-1, K),
                    edge_index[1].astype(jnp.int32).reshape(-1, K)], axis=1)
    zd = jnp.zeros((RB, D), jnp.float32)
    ones = jnp.ones((K, D), jnp.float32)
    b1r = b1.reshape(1, D)
    b2r = b2.reshape(1, D)
    b3r = b3.reshape(1, D)
    bgr = bg.reshape(1, -1)

    pc = _counts(ones, ei, zd).reshape(NC, N_NODES, D)
    p1 = _agg(x, ei, zd).reshape(NC, N_NODES, D)
    h1, recip = pl.pallas_call(
        _tc_layer1,
        out_shape=(jax.ShapeDtypeStruct((N_NODES, D), jnp.float32),
                   jax.ShapeDtypeStruct((N_NODES, D), jnp.float32)),
    )(p1, pc, x, W1_l, W1_r, b1r)

    p2 = _agg(h1, ei, zd).reshape(NC, N_NODES, D)
    h2 = pl.pallas_call(
        _tc_layer,
        out_shape=jax.ShapeDtypeStruct((N_NODES, D), jnp.float32),
    )(p2, recip, h1, W2_l, W2_r, b2r)

    p3 = _agg(h2, ei, zd).reshape(NC, N_NODES, D)
    out = pl.pallas_call(
        _tc_layer3,
        out_shape=jax.ShapeDtypeStruct((1, Wg.shape[1]), jnp.float32),
    )(p3, recip, h2, W3_l, W3_r, b3r, Wg, bgr)
    return out


# speedup vs baseline: 7.5890x; 1.0667x over previous
import jax
import jax.numpy as jnp
from jax import lax
from jax.experimental import pallas as pl
from jax.experimental.pallas import tpu as pltpu
from jax.experimental.pallas import tpu_sc as plsc

N_NODES = 10000
N_EDGES = 320000
D = 128

NC = 2
NS = 16
NW = NC * NS
EPW = N_EDGES // NW
K = 125
CHUNKS = EPW // K
RB = 80
RCHUNKS = N_NODES // RB

_mesh = plsc.VectorSubcoreMesh(core_axis_name="c", subcore_axis_name="s")


def _make_agg(with_gather: bool):
    NB = 2
    scratch = (
        [pltpu.VMEM((2, K), jnp.int32) for _ in range(NB)]
        + [pltpu.VMEM((K, D), jnp.float32) for _ in range(NB)]
        + [
            pltpu.VMEM((RB, D), jnp.float32),
            pltpu.VMEM_SHARED((N_NODES, D), jnp.float32),
        ]
        + [pltpu.SemaphoreType.DMA for _ in range(2 * NB + 1)]
    )

    def body(h_hbm, ei_hbm, zd_hbm, *rest):
        p_hbm = rest[0]
        idx = rest[1:1 + NB]
        rows = rest[1 + NB:1 + 2 * NB]
        zbuf, acc = rest[1 + 2 * NB:3 + 2 * NB]
        sems = rest[3 + 2 * NB:]
        sab = sems[:NB]
        scs = sems[NB:2 * NB]
        sz = sems[2 * NB]
        c = lax.axis_index("c")
        s = lax.axis_index("s")
        wid = c * NS + s

        pltpu.sync_copy(zd_hbm, zbuf)
        if not with_gather:
            for r in rows:
                pltpu.sync_copy(h_hbm.at[pl.ds(0, K)], r)

        @pl.loop(s, RCHUNKS, step=NS)
        def _(j):
            rb = pl.multiple_of(j * RB, 8)
            pltpu.sync_copy(zbuf, acc.at[pl.ds(rb, RB)])

        plsc.subcore_barrier()

        cbase = wid * CHUNKS
        OUTER = CHUNKS // NB

        @pl.loop(0, OUTER)
        def _(g):
            jb = cbase + NB * g

            @pl.when(g > 0)
            def _():
                pltpu.make_async_copy(rows[0], acc.at[idx[0].at[1]],
                                      scs[0]).wait()

            aA = pltpu.async_copy(ei_hbm.at[jb], idx[0], sab[0])

            @pl.when(g > 0)
            def _():
                pltpu.make_async_copy(rows[1], acc.at[idx[1].at[1]],
                                      scs[1]).wait()

            aB = pltpu.async_copy(ei_hbm.at[jb + 1], idx[1], sab[1])
            if with_gather:
                aA.wait()
                gA = pltpu.async_copy(h_hbm.at[idx[0].at[0]], rows[0],
                                      sab[0])
                aB.wait()
                gB = pltpu.async_copy(h_hbm.at[idx[1].at[0]], rows[1],
                                      sab[1])
                gA.wait()
                pltpu.async_copy(rows[0], acc.at[idx[0].at[1]], scs[0],
                                 add=True)
                gB.wait()
                pltpu.async_copy(rows[1], acc.at[idx[1].at[1]], scs[1],
                                 add=True)
            else:
                aA.wait()
                pltpu.async_copy(rows[0], acc.at[idx[0].at[1]], scs[0],
                                 add=True)
                aB.wait()
                pltpu.async_copy(rows[1], acc.at[idx[1].at[1]], scs[1],
                                 add=True)

        for b in range(NB):
            pltpu.make_async_copy(rows[b], acc.at[idx[b].at[1]],
                                  scs[b]).wait()

        plsc.subcore_barrier()

        obase = c * N_NODES

        @pl.loop(s, RCHUNKS, step=NS)
        def _(j):
            rb = pl.multiple_of(j * RB, 8)
            ob = pl.multiple_of(obase + j * RB, 8)
            pltpu.sync_copy(acc.at[pl.ds(rb, RB)], zbuf)
            pltpu.sync_copy(zbuf, p_hbm.at[pl.ds(ob, RB)])

    return pl.kernel(
        body,
        out_type=jax.ShapeDtypeStruct((NC * N_NODES, D), jnp.float32),
        mesh=_mesh, scratch_types=scratch)


_agg = _make_agg(True)
_counts = _make_agg(False)


def _tc_layer1(p_ref, pc_ref, h_ref, wl_ref, wr_ref, b_ref,
               out_ref, recip_ref):
    cnt = pc_ref[0, :, 0:1] + pc_ref[1, :, 0:1]
    recip = 1.0 / jnp.maximum(cnt, 1.0)
    recip_b = jnp.broadcast_to(recip, (N_NODES, D))
    recip_ref[...] = recip_b
    mean = (p_ref[0] + p_ref[1]) * recip_b
    acc = jnp.dot(mean, wl_ref[...], preferred_element_type=jnp.float32)
    acc += jnp.dot(h_ref[...], wr_ref[...], preferred_element_type=jnp.float32)
    out_ref[...] = jnp.maximum(acc + b_ref[...], 0.0)


def _tc_layer(p_ref, recip_ref, h_ref, wl_ref, wr_ref, b_ref, out_ref):
    mean = (p_ref[0] + p_ref[1]) * recip_ref[...]
    acc = jnp.dot(mean, wl_ref[...], preferred_element_type=jnp.float32)
    acc += jnp.dot(h_ref[...], wr_ref[...], preferred_element_type=jnp.float32)
    out_ref[...] = jnp.maximum(acc + b_ref[...], 0.0)


def _tc_layer3(p_ref, recip_ref, h_ref, wl_ref, wr_ref, b_ref,
               wg_ref, bg_ref, out_ref):
    mean = (p_ref[0] + p_ref[1]) * recip_ref[...]
    acc = jnp.dot(mean, wl_ref[...], preferred_element_type=jnp.float32)
    acc += jnp.dot(h_ref[...], wr_ref[...], preferred_element_type=jnp.float32)
    h3 = jnp.maximum(acc + b_ref[...], 0.0)
    g = jnp.mean(h3, axis=0, keepdims=True)
    z = jnp.dot(g, wg_ref[...], preferred_element_type=jnp.float32) + bg_ref[...]
    out_ref[...] = 1.0 / (1.0 + jnp.exp(-z))


def kernel(x, edge_index, W1_l, W1_r, b1, W2_l, W2_r, b2, W3_l, W3_r, b3,
           Wg, bg):
    ei = jnp.stack([edge_index[0].astype(jnp.int32).reshape(-1, K),
                    edge_index[1].astype(jnp.int32).reshape(-1, K)], axis=1)
    zd = jnp.zeros((RB, D), jnp.float32)
    ones = jnp.ones((K, D), jnp.float32)
    b1r = b1.reshape(1, D)
    b2r = b2.reshape(1, D)
    b3r = b3.reshape(1, D)
    bgr = bg.reshape(1, -1)

    pc = _counts(ones, ei, zd).reshape(NC, N_NODES, D)
    p1 = _agg(x, ei, zd).reshape(NC, N_NODES, D)
    h1, recip = pl.pallas_call(
        _tc_layer1,
        out_shape=(jax.ShapeDtypeStruct((N_NODES, D), jnp.float32),
                   jax.ShapeDtypeStruct((N_NODES, D), jnp.float32)),
    )(p1, pc, x, W1_l, W1_r, b1r)

    p2 = _agg(h1, ei, zd).reshape(NC, N_NODES, D)
    h2 = pl.pallas_call(
        _tc_layer,
        out_shape=jax.ShapeDtypeStruct((N_NODES, D), jnp.float32),
    )(p2, recip, h1, W2_l, W2_r, b2r)

    p3 = _agg(h2, ei, zd).reshape(NC, N_NODES, D)
    out = pl.pallas_call(
        _tc_layer3,
        out_shape=jax.ShapeDtypeStruct((1, Wg.shape[1]), jnp.float32),
    )(p3, recip, h2, W3_l, W3_r, b3r, Wg, bgr)
    return out


# speedup vs baseline: 7.5987x; 1.0013x over previous
import jax
import jax.numpy as jnp
from jax import lax
from jax.experimental import pallas as pl
from jax.experimental.pallas import tpu as pltpu
from jax.experimental.pallas import tpu_sc as plsc

N_NODES = 10000
N_EDGES = 320000
D = 128

NC = 2
NS = 16
NW = NC * NS
EPW = N_EDGES // NW
K = 125
CHUNKS = EPW // K
RB = 80
RCHUNKS = N_NODES // RB

_mesh = plsc.VectorSubcoreMesh(core_axis_name="c", subcore_axis_name="s")


def _make_agg(with_gather: bool):
    NB = 2
    scratch = (
        [pltpu.VMEM((2, K), jnp.int32) for _ in range(NB)]
        + [pltpu.VMEM((K, D), jnp.float32) for _ in range(NB)]
        + [
            pltpu.VMEM((RB, D), jnp.float32),
            pltpu.VMEM_SHARED((N_NODES, D), jnp.float32),
        ]
        + [pltpu.SemaphoreType.DMA for _ in range(2 * NB)]
    )

    def body(h_hbm, ei_hbm, zd_hbm, *rest):
        p_hbm = rest[0]
        idx = rest[1:1 + NB]
        rows = rest[1 + NB:1 + 2 * NB]
        zbuf, acc = rest[1 + 2 * NB:3 + 2 * NB]
        sems = rest[3 + 2 * NB:]
        sab = sems[:NB]
        scs = sems[NB:2 * NB]
        c = lax.axis_index("c")
        s = lax.axis_index("s")
        wid = c * NS + s

        pltpu.sync_copy(zd_hbm, zbuf)
        if not with_gather:
            for r in rows:
                pltpu.sync_copy(h_hbm.at[pl.ds(0, K)], r)

        @pl.loop(s, RCHUNKS, step=NS)
        def _(j):
            rb = pl.multiple_of(j * RB, 8)
            pltpu.sync_copy(zbuf, acc.at[pl.ds(rb, RB)])

        plsc.subcore_barrier()

        cbase = wid * CHUNKS
        OUTER = CHUNKS // NB

        @pl.loop(0, OUTER)
        def _(g):
            jb = cbase + NB * g

            @pl.when(g > 0)
            def _():
                pltpu.make_async_copy(rows[0], acc.at[idx[0].at[1]],
                                      scs[0]).wait()

            aA = pltpu.async_copy(ei_hbm.at[jb], idx[0], sab[0])

            @pl.when(g > 0)
            def _():
                pltpu.make_async_copy(rows[1], acc.at[idx[1].at[1]],
                                      scs[1]).wait()

            aB = pltpu.async_copy(ei_hbm.at[jb + 1], idx[1], sab[1])
            if with_gather:
                aA.wait()
                gA = pltpu.async_copy(h_hbm.at[idx[0].at[0]], rows[0],
                                      sab[0])
                aB.wait()
                gB = pltpu.async_copy(h_hbm.at[idx[1].at[0]], rows[1],
                                      sab[1])
                gA.wait()
                pltpu.async_copy(rows[0], acc.at[idx[0].at[1]], scs[0],
                                 add=True)
                gB.wait()
                pltpu.async_copy(rows[1], acc.at[idx[1].at[1]], scs[1],
                                 add=True)
            else:
                aA.wait()
                pltpu.async_copy(rows[0], acc.at[idx[0].at[1]], scs[0],
                                 add=True)
                aB.wait()
                pltpu.async_copy(rows[1], acc.at[idx[1].at[1]], scs[1],
                                 add=True)

        for b in range(NB):
            pltpu.make_async_copy(rows[b], acc.at[idx[b].at[1]],
                                  scs[b]).wait()

        plsc.subcore_barrier()

        obase = c * N_NODES

        @pl.loop(s, RCHUNKS, step=NS)
        def _(j):
            rb = pl.multiple_of(j * RB, 8)
            ob = pl.multiple_of(obase + j * RB, 8)
            pltpu.sync_copy(acc.at[pl.ds(rb, RB)], zbuf)
            pltpu.sync_copy(zbuf, p_hbm.at[pl.ds(ob, RB)])

    return pl.kernel(
        body,
        out_type=jax.ShapeDtypeStruct((NC * N_NODES, D), jnp.float32),
        mesh=_mesh, scratch_types=scratch)


_agg = _make_agg(True)
_counts = _make_agg(False)


def _tc_layer1(p_ref, pc_ref, h_ref, wl_ref, wr_ref, b_ref,
               out_ref, recip_ref):
    cnt = pc_ref[0, :, 0:1] + pc_ref[1, :, 0:1]
    recip = 1.0 / jnp.maximum(cnt, 1.0)
    recip_b = jnp.broadcast_to(recip, (N_NODES, D))
    recip_ref[...] = recip_b
    mean = (p_ref[0] + p_ref[1]) * recip_b
    acc = jnp.dot(mean, wl_ref[...], preferred_element_type=jnp.float32)
    acc += jnp.dot(h_ref[...], wr_ref[...], preferred_element_type=jnp.float32)
    out_ref[...] = jnp.maximum(acc + b_ref[...], 0.0)


def _tc_layer(p_ref, recip_ref, h_ref, wl_ref, wr_ref, b_ref, out_ref):
    mean = (p_ref[0] + p_ref[1]) * recip_ref[...]
    acc = jnp.dot(mean, wl_ref[...], preferred_element_type=jnp.float32)
    acc += jnp.dot(h_ref[...], wr_ref[...], preferred_element_type=jnp.float32)
    out_ref[...] = jnp.maximum(acc + b_ref[...], 0.0)


def _tc_layer3(p_ref, recip_ref, h_ref, wl_ref, wr_ref, b_ref,
               wg_ref, bg_ref, out_ref):
    mean = (p_ref[0] + p_ref[1]) * recip_ref[...]
    acc = jnp.dot(mean, wl_ref[...], preferred_element_type=jnp.float32)
    acc += jnp.dot(h_ref[...], wr_ref[...], preferred_element_type=jnp.float32)
    h3 = jnp.maximum(acc + b_ref[...], 0.0)
    g = jnp.mean(h3, axis=0, keepdims=True)
    z = jnp.dot(g, wg_ref[...], preferred_element_type=jnp.float32) + bg_ref[...]
    out_ref[...] = 1.0 / (1.0 + jnp.exp(-z))


def kernel(x, edge_index, W1_l, W1_r, b1, W2_l, W2_r, b2, W3_l, W3_r, b3,
           Wg, bg):
    ei = jnp.stack([edge_index[0].astype(jnp.int32).reshape(-1, K),
                    edge_index[1].astype(jnp.int32).reshape(-1, K)], axis=1)
    zd = jnp.zeros((RB, D), jnp.float32)
    ones = jnp.ones((K, D), jnp.float32)
    b1r = b1.reshape(1, D)
    b2r = b2.reshape(1, D)
    b3r = b3.reshape(1, D)
    bgr = bg.reshape(1, -1)

    pc = _counts(ones, ei, zd).reshape(NC, N_NODES, D)
    p1 = _agg(x, ei, zd).reshape(NC, N_NODES, D)
    h1, recip = pl.pallas_call(
        _tc_layer1,
        out_shape=(jax.ShapeDtypeStruct((N_NODES, D), jnp.float32),
                   jax.ShapeDtypeStruct((N_NODES, D), jnp.float32)),
    )(p1, pc, x, W1_l, W1_r, b1r)

    p2 = _agg(h1, ei, zd).reshape(NC, N_NODES, D)
    h2 = pl.pallas_call(
        _tc_layer,
        out_shape=jax.ShapeDtypeStruct((N_NODES, D), jnp.float32),
    )(p2, recip, h1, W2_l, W2_r, b2r)

    p3 = _agg(h2, ei, zd).reshape(NC, N_NODES, D)
    out = pl.pallas_call(
        _tc_layer3,
        out_shape=jax.ShapeDtypeStruct((1, Wg.shape[1]), jnp.float32),
    )(p3, recip, h2, W3_l, W3_r, b3r, Wg, bgr)
    return out
